# baseline (speedup 1.0000x reference)
import jax
import jax.numpy as jnp
from jax import lax
from jax.experimental import pallas as pl
from jax.experimental.pallas import tpu as pltpu
from jax.experimental.pallas import tpu_sc as plsc

VOCAB = 30522
HIDDEN = 768
NUM_LABELS = 2
BATCH = 4096
SEQ = 200

L = 16
NC = 2
NS = 16
NW = NC * NS
ROWS_PER_W = BATCH // NW
LG = ROWS_PER_W // L
VP = VOCAB + 6
ZCOL = VOCAB
TROWS = 3
CHUNK = 8
NCHUNK = SEQ // CHUNK

_PROJ_BLK = 512


def _proj_body(w_ref, emb_ref, out_ref):
    res = lax.dot_general(w_ref[...], emb_ref[...],
                          (((0,), (1,)), ((), ())),
                          preferred_element_type=jnp.float32)
    row = lax.broadcasted_iota(jnp.int32, res.shape, 0)
    out_ref[...] = jnp.where(row == NUM_LABELS, 1.0, res)


def _project_table(emb_table, w_pad):
    grid = (pl.cdiv(VOCAB, _PROJ_BLK),)
    return pl.pallas_call(
        _proj_body,
        grid=grid,
        in_specs=[
            pl.BlockSpec((HIDDEN, TROWS), lambda i: (0, 0)),
            pl.BlockSpec((_PROJ_BLK, HIDDEN), lambda i: (i, 0)),
        ],
        out_specs=pl.BlockSpec((TROWS, _PROJ_BLK), lambda i: (0, i)),
        out_shape=jax.ShapeDtypeStruct((TROWS, VOCAB), jnp.float32),
    )(w_pad, emb_table)


def _lane_bcast(vec, lane):
    idx = jnp.full((L, 1), lane, jnp.int32)
    dnums = lax.GatherDimensionNumbers(
        offset_dims=(), collapsed_slice_dims=(0,), start_index_map=(0,))
    return lax.gather(vec, idx, dnums, slice_sizes=(1,),
                      mode=lax.GatherScatterMode.PROMISE_IN_BOUNDS)


def _sc_body(ids_hbm, mask_hbm, tab_hbm, bias_hbm, out_hbm,
             tabv, idsv, maskv, accv, outv, biasv, sem_tab, sem_io):
    wid = lax.axis_index("s") * NC + lax.axis_index("c")
    row0 = wid * ROWS_PER_W

    tab_dma = pltpu.async_copy(tab_hbm, tabv, sem_tab)
    pltpu.sync_copy(bias_hbm, biasv)
    bias_vec = biasv[...]
    b0 = _lane_bcast(bias_vec, 0)
    b1 = _lane_bcast(bias_vec, 1)

    def fire(c, buf):
        pltpu.async_copy(
            ids_hbm.at[pl.ds(row0, ROWS_PER_W), pl.ds(c * CHUNK, CHUNK)],
            idsv.at[buf], sem_io)
        pltpu.async_copy(
            mask_hbm.at[pl.ds(row0, ROWS_PER_W), pl.ds(c * CHUNK, CHUNK)],
            maskv.at[buf], sem_io)

    def drain(buf):
        pltpu.make_async_copy(
            ids_hbm.at[pl.ds(0, ROWS_PER_W), pl.ds(0, CHUNK)],
            idsv.at[buf], sem_io).wait()
        pltpu.make_async_copy(
            mask_hbm.at[pl.ds(0, ROWS_PER_W), pl.ds(0, CHUNK)],
            maskv.at[buf], sem_io).wait()

    fire(0, 0)
    fire(1, 1)
    tab_dma.wait()

    c1 = jnp.full((L,), VP, jnp.int32)
    c2 = jnp.full((L,), 2 * VP, jnp.int32)
    zcol = jnp.full((L,), ZCOL, jnp.int32)
    rbase = [lax.iota(jnp.int32, L) + lg * L for lg in range(LG)]

    zero = jnp.zeros((L,), jnp.float32)
    for j in range(3 * LG):
        accv[j] = zero

    csis = [jnp.full((L,), si, jnp.int32) for si in range(CHUNK)]

    def consume(buf):
        for lg in range(LG):
            a0 = accv[3 * lg + 0]
            a1 = accv[3 * lg + 1]
            a2 = accv[3 * lg + 2]
            for si in range(CHUNK):
                ids16 = plsc.load_gather(idsv.at[buf], [rbase[lg], csis[si]])
                m16 = plsc.load_gather(maskv.at[buf], [rbase[lg], csis[si]])
                sel = jnp.where(m16 != 0, ids16, zcol)
                g0 = plsc.load_gather(tabv, [sel])
                g1 = plsc.load_gather(tabv, [sel + c1])
                g2 = plsc.load_gather(tabv, [sel + c2])
                a0 = a0 + g0
                a1 = a1 + g1
                a2 = a2 + g2
            accv[3 * lg + 0] = a0
            accv[3 * lg + 1] = a1
            accv[3 * lg + 2] = a2

    def pair_body(p, _):
        c = 2 * p
        drain(0)
        consume(0)

        @pl.when(c + 2 < NCHUNK)
        def _():
            fire(c + 2, 0)
        drain(1)
        consume(1)

        @pl.when(c + 3 < NCHUNK)
        def _():
            fire(c + 3, 1)
        return 0

    lax.fori_loop(0, NCHUNK // 2, pair_body, 0)
    drain(0)
    consume(0)

    for lg in range(LG):
        a0 = accv[3 * lg + 0]
        a1 = accv[3 * lg + 1]
        a2 = accv[3 * lg + 2]
        outv[0, pl.ds(lg * L, L)] = a0 / a2 + b0
        outv[1, pl.ds(lg * L, L)] = a1 / a2 + b1
    pltpu.sync_copy(outv, out_hbm.at[wid])


def _sc_pool(ids, mask, tab_flat, bias_pad):
    mesh = plsc.VectorSubcoreMesh(core_axis_name="c", subcore_axis_name="s",
                                  num_cores=NC, num_subcores=NS)
    f = pl.kernel(
        _sc_body,
        out_type=jax.ShapeDtypeStruct((NW, NUM_LABELS, ROWS_PER_W),
                                      jnp.float32),
        mesh=mesh,
        scratch_types=[
            pltpu.VMEM((TROWS * VP,), jnp.float32),
            pltpu.VMEM((2, ROWS_PER_W, CHUNK), jnp.int32),
            pltpu.VMEM((2, ROWS_PER_W, CHUNK), jnp.int32),
            pltpu.VMEM((3 * LG, L), jnp.float32),
            pltpu.VMEM((NUM_LABELS, ROWS_PER_W), jnp.float32),
            pltpu.VMEM((L,), jnp.float32),
            pltpu.SemaphoreType.DMA,
            pltpu.SemaphoreType.DMA,
        ],
        compiler_params=pltpu.CompilerParams(use_tc_tiling_on_sc=False,
                                             needs_layout_passes=False),
    )
    return f(ids, mask, tab_flat, bias_pad)


@jax.jit
def kernel(input_ids, attention_mask, emb_table, W, b):
    w_pad = jnp.pad(W.astype(jnp.float32), ((0, 0), (0, TROWS - NUM_LABELS)))
    proj = _project_table(emb_table.astype(jnp.float32), w_pad)
    tab_flat = jnp.pad(proj, ((0, 0), (0, VP - VOCAB))).reshape(-1)
    bias_pad = jnp.pad(b.astype(jnp.float32), (0, L - NUM_LABELS))

    out3 = _sc_pool(input_ids.astype(jnp.int32),
                    attention_mask.astype(jnp.int32), tab_flat, bias_pad)
    return out3.transpose(0, 2, 1).reshape(BATCH, NUM_LABELS)


# speedup vs baseline: 1.1770x; 1.1770x over previous
import jax
import jax.numpy as jnp
from jax import lax
from jax.experimental import pallas as pl
from jax.experimental.pallas import tpu as pltpu
from jax.experimental.pallas import tpu_sc as plsc

VOCAB = 30522
HIDDEN = 768
NUM_LABELS = 2
BATCH = 4096
SEQ = 200

L = 16
NC = 2
NS = 16
NW = NC * NS
ROWS_PER_W = BATCH // NW
LG = ROWS_PER_W // L
VP = VOCAB + 6
ZCOL = VOCAB
TROWS = 3
CHUNK = 16
NFULL = SEQ // CHUNK
TAIL = SEQ - NFULL * CHUNK

_PROJ_BLK = 2048


def _proj_body(w_ref, emb_ref, out_ref):
    res = lax.dot_general(w_ref[...], emb_ref[...],
                          (((0,), (1,)), ((), ())),
                          preferred_element_type=jnp.float32)
    out_ref[...] = jnp.concatenate(
        [res, jnp.ones((1, res.shape[1]), jnp.float32)], axis=0)


def _project_table(emb_table, W):
    grid = (pl.cdiv(VOCAB, _PROJ_BLK),)
    return pl.pallas_call(
        _proj_body,
        grid=grid,
        in_specs=[
            pl.BlockSpec((HIDDEN, NUM_LABELS), lambda i: (0, 0)),
            pl.BlockSpec((_PROJ_BLK, HIDDEN), lambda i: (i, 0)),
        ],
        out_specs=pl.BlockSpec((TROWS, _PROJ_BLK), lambda i: (0, i)),
        out_shape=jax.ShapeDtypeStruct((TROWS, VOCAB), jnp.float32),
    )(W, emb_table)


def _lane_bcast(vec, lane):
    idx = jnp.full((L, 1), lane, jnp.int32)
    dnums = lax.GatherDimensionNumbers(
        offset_dims=(), collapsed_slice_dims=(0,), start_index_map=(0,))
    return lax.gather(vec, idx, dnums, slice_sizes=(1,),
                      mode=lax.GatherScatterMode.PROMISE_IN_BOUNDS)


def _sc_body(ids_hbm, mask_hbm, tab_hbm, bias_hbm, out_hbm,
             tabv, idsv, maskv, accv, outv, biasv, sem_tab, sem_io):
    wid = lax.axis_index("s") * NC + lax.axis_index("c")
    row0 = wid * ROWS_PER_W

    tab_dma = pltpu.async_copy(tab_hbm, tabv, sem_tab)
    pltpu.sync_copy(bias_hbm, biasv)
    bias_vec = biasv[...]
    b0 = _lane_bcast(bias_vec, 0)
    b1 = _lane_bcast(bias_vec, 1)

    def fire(c, buf, width):
        pltpu.async_copy(
            ids_hbm.at[pl.ds(row0, ROWS_PER_W), pl.ds(c * CHUNK, width)],
            idsv.at[buf, :, pl.ds(0, width)], sem_io)
        pltpu.async_copy(
            mask_hbm.at[pl.ds(row0, ROWS_PER_W), pl.ds(c * CHUNK, width)],
            maskv.at[buf, :, pl.ds(0, width)], sem_io)

    def drain(buf, width):
        pltpu.make_async_copy(
            ids_hbm.at[pl.ds(0, ROWS_PER_W), pl.ds(0, width)],
            idsv.at[buf, :, pl.ds(0, width)], sem_io).wait()
        pltpu.make_async_copy(
            mask_hbm.at[pl.ds(0, ROWS_PER_W), pl.ds(0, width)],
            maskv.at[buf, :, pl.ds(0, width)], sem_io).wait()

    fire(0, 0, CHUNK)
    fire(1, 1, CHUNK)
    tab_dma.wait()

    c1 = jnp.full((L,), VP, jnp.int32)
    c2 = jnp.full((L,), 2 * VP, jnp.int32)
    zcol = jnp.full((L,), ZCOL, jnp.int32)
    iota = lax.iota(jnp.int32, L)
    rbase = [iota + lg * L for lg in range(LG)]

    zero = jnp.zeros((L,), jnp.float32)
    for j in range(3 * LG):
        accv[j] = zero

    csis = [jnp.full((L,), si, jnp.int32) for si in range(CHUNK)]

    def consume(buf, width):
        for lg in range(LG):
            a0 = accv[3 * lg + 0]
            a1 = accv[3 * lg + 1]
            a2 = accv[3 * lg + 2]
            for si in range(width):
                ids16 = plsc.load_gather(idsv.at[buf], [rbase[lg], csis[si]])
                m16 = plsc.load_gather(maskv.at[buf], [rbase[lg], csis[si]])
                sel = jnp.where(m16 != 0, ids16, zcol)
                g0 = plsc.load_gather(tabv, [sel])
                g1 = plsc.load_gather(tabv, [sel + c1])
                g2 = plsc.load_gather(tabv, [sel + c2])
                a0 = a0 + g0
                a1 = a1 + g1
                a2 = a2 + g2
            accv[3 * lg + 0] = a0
            accv[3 * lg + 1] = a1
            accv[3 * lg + 2] = a2

    def pair_body(p, _):
        c = 2 * p
        drain(0, CHUNK)
        consume(0, CHUNK)

        @pl.when(c + 2 < NFULL)
        def _():
            fire(c + 2, 0, CHUNK)

        @pl.when(c + 2 == NFULL)
        def _():
            fire(c + 2, 0, TAIL)
        drain(1, CHUNK)
        consume(1, CHUNK)

        @pl.when(c + 3 < NFULL)
        def _():
            fire(c + 3, 1, CHUNK)
        return 0

    lax.fori_loop(0, NFULL // 2, pair_body, 0)
    drain(0, TAIL)
    consume(0, TAIL)

    col0 = jnp.zeros((L,), jnp.int32)
    col1 = jnp.ones((L,), jnp.int32)
    for lg in range(LG):
        a0 = accv[3 * lg + 0]
        a1 = accv[3 * lg + 1]
        a2 = accv[3 * lg + 2]
        plsc.store_scatter(outv, [rbase[lg], col0], a0 / a2 + b0)
        plsc.store_scatter(outv, [rbase[lg], col1], a1 / a2 + b1)
    pltpu.sync_copy(outv, out_hbm.at[pl.ds(row0, ROWS_PER_W)])


def _sc_pool(ids, mask, tab_flat, bias_pad):
    mesh = plsc.VectorSubcoreMesh(core_axis_name="c", subcore_axis_name="s",
                                  num_cores=NC, num_subcores=NS)
    f = pl.kernel(
        _sc_body,
        out_type=jax.ShapeDtypeStruct((BATCH, NUM_LABELS), jnp.float32),
        mesh=mesh,
        scratch_types=[
            pltpu.VMEM((TROWS * VP,), jnp.float32),
            pltpu.VMEM((2, ROWS_PER_W, CHUNK), jnp.int32),
            pltpu.VMEM((2, ROWS_PER_W, CHUNK), jnp.int32),
            pltpu.VMEM((3 * LG, L), jnp.float32),
            pltpu.VMEM((ROWS_PER_W, NUM_LABELS), jnp.float32),
            pltpu.VMEM((L,), jnp.float32),
            pltpu.SemaphoreType.DMA,
            pltpu.SemaphoreType.DMA,
        ],
        compiler_params=pltpu.CompilerParams(use_tc_tiling_on_sc=False,
                                             needs_layout_passes=False),
    )
    return f(ids, mask, tab_flat, bias_pad)


@jax.jit
def kernel(input_ids, attention_mask, emb_table, W, b):
    proj = _project_table(emb_table.astype(jnp.float32),
                          W.astype(jnp.float32))
    tab_flat = jnp.pad(proj, ((0, 0), (0, VP - VOCAB))).reshape(-1)
    bias_pad = jnp.pad(b.astype(jnp.float32), (0, L - NUM_LABELS))

    return _sc_pool(input_ids.astype(jnp.int32),
                    attention_mask.astype(jnp.int32), tab_flat, bias_pad)


# speedup vs baseline: 1.2729x; 1.0815x over previous
import jax
import jax.numpy as jnp
from jax import lax
from jax.experimental import pallas as pl
from jax.experimental.pallas import tpu as pltpu
from jax.experimental.pallas import tpu_sc as plsc

VOCAB = 30522
HIDDEN = 768
NUM_LABELS = 2
BATCH = 4096
SEQ = 200

L = 16
NC = 2
NS = 16
NW = NC * NS
ROWS_PER_W = BATCH // NW
LG = ROWS_PER_W // L
VP = VOCAB + 6
ZCOL = VOCAB
TROWS = 2
CHUNK = 16
NFULL = SEQ // CHUNK
TAIL = SEQ - NFULL * CHUNK

_PROJ_BLK = 2048


def _proj_body(w_ref, emb_ref, out_ref):
    out_ref[...] = lax.dot_general(w_ref[...], emb_ref[...],
                                   (((0,), (1,)), ((), ())),
                                   preferred_element_type=jnp.float32)


def _project_table(emb_table, W):
    grid = (pl.cdiv(VOCAB, _PROJ_BLK),)
    return pl.pallas_call(
        _proj_body,
        grid=grid,
        in_specs=[
            pl.BlockSpec((HIDDEN, NUM_LABELS), lambda i: (0, 0)),
            pl.BlockSpec((_PROJ_BLK, HIDDEN), lambda i: (i, 0)),
        ],
        out_specs=pl.BlockSpec((TROWS, _PROJ_BLK), lambda i: (0, i)),
        out_shape=jax.ShapeDtypeStruct((TROWS, VOCAB), jnp.float32),
    )(W, emb_table)


def _lane_bcast(vec, lane):
    idx = jnp.full((L, 1), lane, jnp.int32)
    dnums = lax.GatherDimensionNumbers(
        offset_dims=(), collapsed_slice_dims=(0,), start_index_map=(0,))
    return lax.gather(vec, idx, dnums, slice_sizes=(1,),
                      mode=lax.GatherScatterMode.PROMISE_IN_BOUNDS)


def _sc_body(ids_hbm, mask_hbm, tab_hbm, bias_hbm, out_hbm,
             tabv, idsv, maskv, accv, outv, biasv, sem_tab, sem_io):
    wid = lax.axis_index("s") * NC + lax.axis_index("c")
    row0 = wid * ROWS_PER_W

    tab_dma = pltpu.async_copy(tab_hbm, tabv, sem_tab)
    pltpu.sync_copy(bias_hbm, biasv)
    bias_vec = biasv[...]
    b0 = _lane_bcast(bias_vec, 0)
    b1 = _lane_bcast(bias_vec, 1)

    def fire(c, buf, width):
        pltpu.async_copy(
            ids_hbm.at[pl.ds(row0, ROWS_PER_W), pl.ds(c * CHUNK, width)],
            idsv.at[buf, :, pl.ds(0, width)], sem_io)
        pltpu.async_copy(
            mask_hbm.at[pl.ds(row0, ROWS_PER_W), pl.ds(c * CHUNK, width)],
            maskv.at[buf, :, pl.ds(0, width)], sem_io)

    def drain(buf, width):
        pltpu.make_async_copy(
            ids_hbm.at[pl.ds(0, ROWS_PER_W), pl.ds(0, width)],
            idsv.at[buf, :, pl.ds(0, width)], sem_io).wait()
        pltpu.make_async_copy(
            mask_hbm.at[pl.ds(0, ROWS_PER_W), pl.ds(0, width)],
            maskv.at[buf, :, pl.ds(0, width)], sem_io).wait()

    fire(0, 0, CHUNK)
    fire(1, 1, CHUNK)
    tab_dma.wait()

    c1 = jnp.full((L,), VP, jnp.int32)
    zcol = jnp.full((L,), ZCOL, jnp.int32)
    iota = lax.iota(jnp.int32, L)
    rbase = [iota + lg * L for lg in range(LG)]

    zero = jnp.zeros((L,), jnp.float32)
    for j in range(3 * LG):
        accv[j] = zero

    csis = [jnp.full((L,), si, jnp.int32) for si in range(CHUNK)]

    def consume(buf, width):
        for lg in range(LG):
            a0 = accv[3 * lg + 0]
            a1 = accv[3 * lg + 1]
            a2 = accv[3 * lg + 2]
            for si in range(width):
                ids16 = plsc.load_gather(idsv.at[buf], [rbase[lg], csis[si]])
                m16 = plsc.load_gather(maskv.at[buf], [rbase[lg], csis[si]])
                sel = jnp.where(m16 != 0, ids16, zcol)
                g0 = plsc.load_gather(tabv, [sel])
                g1 = plsc.load_gather(tabv, [sel + c1])
                a0 = a0 + g0
                a1 = a1 + g1
                a2 = a2 + m16.astype(jnp.float32)
            accv[3 * lg + 0] = a0
            accv[3 * lg + 1] = a1
            accv[3 * lg + 2] = a2

    def pair_body(p, _):
        c = 2 * p
        drain(0, CHUNK)
        consume(0, CHUNK)

        @pl.when(c + 2 < NFULL)
        def _():
            fire(c + 2, 0, CHUNK)

        @pl.when(c + 2 == NFULL)
        def _():
            fire(c + 2, 0, TAIL)
        drain(1, CHUNK)
        consume(1, CHUNK)

        @pl.when(c + 3 < NFULL)
        def _():
            fire(c + 3, 1, CHUNK)
        return 0

    lax.fori_loop(0, NFULL // 2, pair_body, 0)
    drain(0, TAIL)
    consume(0, TAIL)

    col0 = jnp.zeros((L,), jnp.int32)
    col1 = jnp.ones((L,), jnp.int32)
    for lg in range(LG):
        a0 = accv[3 * lg + 0]
        a1 = accv[3 * lg + 1]
        a2 = accv[3 * lg + 2]
        plsc.store_scatter(outv, [rbase[lg], col0], a0 / a2 + b0)
        plsc.store_scatter(outv, [rbase[lg], col1], a1 / a2 + b1)
    pltpu.sync_copy(outv, out_hbm.at[pl.ds(row0, ROWS_PER_W)])


def _sc_pool(ids, mask, tab_flat, bias_pad):
    mesh = plsc.VectorSubcoreMesh(core_axis_name="c", subcore_axis_name="s",
                                  num_cores=NC, num_subcores=NS)
    f = pl.kernel(
        _sc_body,
        out_type=jax.ShapeDtypeStruct((BATCH, NUM_LABELS), jnp.float32),
        mesh=mesh,
        scratch_types=[
            pltpu.VMEM((TROWS * VP,), jnp.float32),
            pltpu.VMEM((2, ROWS_PER_W, CHUNK), jnp.int32),
            pltpu.VMEM((2, ROWS_PER_W, CHUNK), jnp.int32),
            pltpu.VMEM((3 * LG, L), jnp.float32),
            pltpu.VMEM((ROWS_PER_W, NUM_LABELS), jnp.float32),
            pltpu.VMEM((L,), jnp.float32),
            pltpu.SemaphoreType.DMA,
            pltpu.SemaphoreType.DMA,
        ],
        compiler_params=pltpu.CompilerParams(use_tc_tiling_on_sc=False,
                                             needs_layout_passes=False),
    )
    return f(ids, mask, tab_flat, bias_pad)


@jax.jit
def kernel(input_ids, attention_mask, emb_table, W, b):
    proj = _project_table(emb_table.astype(jnp.float32),
                          W.astype(jnp.float32))
    tab_flat = jnp.pad(proj, ((0, 0), (0, VP - VOCAB))).reshape(-1)
    bias_pad = jnp.pad(b.astype(jnp.float32), (0, L - NUM_LABELS))

    return _sc_pool(input_ids.astype(jnp.int32),
                    attention_mask.astype(jnp.int32), tab_flat, bias_pad)


# speedup vs baseline: 1.3838x; 1.0871x over previous
import jax
import jax.numpy as jnp
from jax import lax
from jax.experimental import pallas as pl
from jax.experimental.pallas import tpu as pltpu
from jax.experimental.pallas import tpu_sc as plsc

VOCAB = 30522
HIDDEN = 768
NUM_LABELS = 2
BATCH = 4096
SEQ = 200

L = 16
NC = 2
NS = 16
NW = NC * NS
ROWS_PER_W = BATCH // NW
LG = ROWS_PER_W // L
VP = VOCAB + 6
ZCOL = VOCAB
TROWS = 2
CHUNK = 16
NFULL = SEQ // CHUNK
TAIL = SEQ - NFULL * CHUNK

_PROJ_BLK = 2048


def _proj_body(w_ref, emb_ref, out_ref):
    out_ref[...] = lax.dot_general(w_ref[...], emb_ref[...],
                                   (((0,), (1,)), ((), ())),
                                   preferred_element_type=jnp.float32)


def _project_table(emb_table, W):
    grid = (pl.cdiv(VOCAB, _PROJ_BLK),)
    return pl.pallas_call(
        _proj_body,
        grid=grid,
        in_specs=[
            pl.BlockSpec((HIDDEN, NUM_LABELS), lambda i: (0, 0)),
            pl.BlockSpec((_PROJ_BLK, HIDDEN), lambda i: (i, 0)),
        ],
        out_specs=pl.BlockSpec((TROWS, _PROJ_BLK), lambda i: (0, i)),
        out_shape=jax.ShapeDtypeStruct((TROWS, VOCAB), jnp.float32),
    )(W, emb_table)


def _lane_bcast(vec, lane):
    idx = jnp.full((L, 1), lane, jnp.int32)
    dnums = lax.GatherDimensionNumbers(
        offset_dims=(), collapsed_slice_dims=(0,), start_index_map=(0,))
    return lax.gather(vec, idx, dnums, slice_sizes=(1,),
                      mode=lax.GatherScatterMode.PROMISE_IN_BOUNDS)


def _sc_body(ids_hbm, mask_hbm, tab_hbm, bias_hbm, out_hbm,
             tabv, idsv, maskv, outv, biasv, sem_tab, sem_io):
    wid = lax.axis_index("s") * NC + lax.axis_index("c")
    row0 = wid * ROWS_PER_W

    tab_dma = pltpu.async_copy(tab_hbm, tabv, sem_tab)
    ids_dma = pltpu.async_copy(ids_hbm.at[pl.ds(row0, ROWS_PER_W)],
                               idsv, sem_io)
    mask_dma = pltpu.async_copy(mask_hbm.at[pl.ds(row0, ROWS_PER_W)],
                                maskv, sem_io)
    pltpu.sync_copy(bias_hbm, biasv)
    bias_vec = biasv[...]
    b0 = _lane_bcast(bias_vec, 0)
    b1 = _lane_bcast(bias_vec, 1)

    c1 = jnp.full((L,), VP, jnp.int32)
    zcol = jnp.full((L,), ZCOL, jnp.int32)
    iota = lax.iota(jnp.int32, L)
    rbase = [iota + lg * L for lg in range(LG)]
    col0 = jnp.zeros((L,), jnp.int32)
    col1 = jnp.ones((L,), jnp.int32)
    zero = jnp.zeros((L,), jnp.float32)

    tab_dma.wait()
    ids_dma.wait()
    mask_dma.wait()

    for lg in range(LG):
        def body(si, accs):
            a0, a1, a2 = accs
            csi = jnp.full((L,), si, jnp.int32)
            ids16 = plsc.load_gather(idsv, [rbase[lg], csi])
            m16 = plsc.load_gather(maskv, [rbase[lg], csi])
            sel = jnp.where(m16 != 0, ids16, zcol)
            g0 = plsc.load_gather(tabv, [sel])
            g1 = plsc.load_gather(tabv, [sel + c1])
            return (a0 + g0, a1 + g1, a2 + m16.astype(jnp.float32))
        a0, a1, a2 = lax.fori_loop(0, SEQ, body, (zero, zero, zero),
                                   unroll=8)
        plsc.store_scatter(outv, [rbase[lg], col0], a0 / a2 + b0)
        plsc.store_scatter(outv, [rbase[lg], col1], a1 / a2 + b1)
    pltpu.sync_copy(outv, out_hbm.at[pl.ds(row0, ROWS_PER_W)])


def _sc_pool(ids, mask, tab_flat, bias_pad):
    mesh = plsc.VectorSubcoreMesh(core_axis_name="c", subcore_axis_name="s",
                                  num_cores=NC, num_subcores=NS)
    f = pl.kernel(
        _sc_body,
        out_type=jax.ShapeDtypeStruct((BATCH, NUM_LABELS), jnp.float32),
        mesh=mesh,
        scratch_types=[
            pltpu.VMEM((TROWS * VP,), jnp.float32),
            pltpu.VMEM((ROWS_PER_W, SEQ), jnp.int32),
            pltpu.VMEM((ROWS_PER_W, SEQ), jnp.int32),
            pltpu.VMEM((ROWS_PER_W, NUM_LABELS), jnp.float32),
            pltpu.VMEM((L,), jnp.float32),
            pltpu.SemaphoreType.DMA,
            pltpu.SemaphoreType.DMA,
        ],
        compiler_params=pltpu.CompilerParams(use_tc_tiling_on_sc=False,
                                             needs_layout_passes=False),
    )
    return f(ids, mask, tab_flat, bias_pad)


@jax.jit
def kernel(input_ids, attention_mask, emb_table, W, b):
    proj = _project_table(emb_table.astype(jnp.float32),
                          W.astype(jnp.float32))
    tab_flat = jnp.pad(proj, ((0, 0), (0, VP - VOCAB))).reshape(-1)
    bias_pad = jnp.pad(b.astype(jnp.float32), (0, L - NUM_LABELS))

    return _sc_pool(input_ids.astype(jnp.int32),
                    attention_mask.astype(jnp.int32), tab_flat, bias_pad)


# speedup vs baseline: 1.3843x; 1.0004x over previous
import jax
import jax.numpy as jnp
from jax import lax
from jax.experimental import pallas as pl
from jax.experimental.pallas import tpu as pltpu
from jax.experimental.pallas import tpu_sc as plsc

VOCAB = 30522
HIDDEN = 768
NUM_LABELS = 2
BATCH = 4096
SEQ = 200

L = 16
NC = 2
NS = 16
NW = NC * NS
ROWS_PER_W = BATCH // NW
LG = ROWS_PER_W // L
VP = VOCAB + 6
ZCOL = VOCAB
TROWS = 2
CHUNK = 16
NFULL = SEQ // CHUNK
TAIL = SEQ - NFULL * CHUNK

_PROJ_BLK = 4096


def _proj_body(w_ref, emb_ref, out_ref):
    out_ref[...] = lax.dot_general(w_ref[...], emb_ref[...],
                                   (((0,), (1,)), ((), ())),
                                   preferred_element_type=jnp.float32)


def _project_table(emb_table, W):
    grid = (pl.cdiv(VOCAB, _PROJ_BLK),)
    return pl.pallas_call(
        _proj_body,
        grid=grid,
        in_specs=[
            pl.BlockSpec((HIDDEN, NUM_LABELS), lambda i: (0, 0)),
            pl.BlockSpec((_PROJ_BLK, HIDDEN), lambda i: (i, 0)),
        ],
        out_specs=pl.BlockSpec((TROWS, _PROJ_BLK), lambda i: (0, i)),
        out_shape=jax.ShapeDtypeStruct((TROWS, VOCAB), jnp.float32),
    )(W, emb_table)


def _lane_bcast(vec, lane):
    idx = jnp.full((L, 1), lane, jnp.int32)
    dnums = lax.GatherDimensionNumbers(
        offset_dims=(), collapsed_slice_dims=(0,), start_index_map=(0,))
    return lax.gather(vec, idx, dnums, slice_sizes=(1,),
                      mode=lax.GatherScatterMode.PROMISE_IN_BOUNDS)


def _sc_body(ids_hbm, mask_hbm, tab_hbm, bias_hbm, out_hbm,
             tabv, idsv, maskv, outv, biasv, sem_tab, sem_io):
    wid = lax.axis_index("s") * NC + lax.axis_index("c")
    row0 = wid * ROWS_PER_W

    half = ROWS_PER_W // 2
    tab_dma = pltpu.async_copy(tab_hbm, tabv, sem_tab)
    io_dmas = []
    for h in range(2):
        io_dmas.append(pltpu.async_copy(
            ids_hbm.at[pl.ds(row0 + h * half, half)],
            idsv.at[pl.ds(h * half, half)], sem_io))
        io_dmas.append(pltpu.async_copy(
            mask_hbm.at[pl.ds(row0 + h * half, half)],
            maskv.at[pl.ds(h * half, half)], sem_io))
    pltpu.sync_copy(bias_hbm, biasv)
    bias_vec = biasv[...]
    b0 = _lane_bcast(bias_vec, 0)
    b1 = _lane_bcast(bias_vec, 1)

    c1 = jnp.full((L,), VP, jnp.int32)
    zcol = jnp.full((L,), ZCOL, jnp.int32)
    iota = lax.iota(jnp.int32, L)
    rbase = [iota + lg * L for lg in range(LG)]
    col0 = jnp.zeros((L,), jnp.int32)
    col1 = jnp.ones((L,), jnp.int32)
    zero = jnp.zeros((L,), jnp.float32)

    tab_dma.wait()
    io_dmas[0].wait()
    io_dmas[1].wait()

    for lg in range(LG):
        if lg == LG // 2:
            io_dmas[2].wait()
            io_dmas[3].wait()
        def body(si, accs):
            a0, a1, a2 = accs
            csi = jnp.full((L,), si, jnp.int32)
            ids16 = plsc.load_gather(idsv, [rbase[lg], csi])
            m16 = plsc.load_gather(maskv, [rbase[lg], csi])
            sel = jnp.where(m16 != 0, ids16, zcol)
            g0 = plsc.load_gather(tabv, [sel])
            g1 = plsc.load_gather(tabv, [sel + c1])
            return (a0 + g0, a1 + g1, a2 + m16.astype(jnp.float32))
        a0, a1, a2 = lax.fori_loop(0, SEQ, body, (zero, zero, zero),
                                   unroll=8)
        plsc.store_scatter(outv, [rbase[lg], col0], a0 / a2 + b0)
        plsc.store_scatter(outv, [rbase[lg], col1], a1 / a2 + b1)
    pltpu.sync_copy(outv, out_hbm.at[pl.ds(row0, ROWS_PER_W)])


def _sc_pool(ids, mask, tab_flat, bias_pad):
    mesh = plsc.VectorSubcoreMesh(core_axis_name="c", subcore_axis_name="s",
                                  num_cores=NC, num_subcores=NS)
    f = pl.kernel(
        _sc_body,
        out_type=jax.ShapeDtypeStruct((BATCH, NUM_LABELS), jnp.float32),
        mesh=mesh,
        scratch_types=[
            pltpu.VMEM((TROWS * VP,), jnp.float32),
            pltpu.VMEM((ROWS_PER_W, SEQ), jnp.int32),
            pltpu.VMEM((ROWS_PER_W, SEQ), jnp.int32),
            pltpu.VMEM((ROWS_PER_W, NUM_LABELS), jnp.float32),
            pltpu.VMEM((L,), jnp.float32),
            pltpu.SemaphoreType.DMA,
            pltpu.SemaphoreType.DMA,
        ],
        compiler_params=pltpu.CompilerParams(use_tc_tiling_on_sc=False,
                                             needs_layout_passes=False),
    )
    return f(ids, mask, tab_flat, bias_pad)


@jax.jit
def kernel(input_ids, attention_mask, emb_table, W, b):
    proj = _project_table(emb_table.astype(jnp.float32),
                          W.astype(jnp.float32))
    tab_flat = jnp.pad(proj, ((0, 0), (0, VP - VOCAB))).reshape(-1)
    bias_pad = jnp.pad(b.astype(jnp.float32), (0, L - NUM_LABELS))

    return _sc_pool(input_ids.astype(jnp.int32),
                    attention_mask.astype(jnp.int32), tab_flat, bias_pad)


# speedup vs baseline: 1.4631x; 1.0569x over previous
import jax
import jax.numpy as jnp
from jax import lax
from jax.experimental import pallas as pl
from jax.experimental.pallas import tpu as pltpu
from jax.experimental.pallas import tpu_sc as plsc

VOCAB = 30522
HIDDEN = 768
NUM_LABELS = 2
BATCH = 4096
SEQ = 200

L = 16
NC = 2
NS = 16
NW = NC * NS
ROWS_PER_W = BATCH // NW
LG = ROWS_PER_W // L
VP = VOCAB + 6
ZCOL = VOCAB
TROWS = 2
CHUNK = 16
NFULL = SEQ // CHUNK
TAIL = SEQ - NFULL * CHUNK

_PROJ_BLK = 4096


def _rne_bf16_bits(x_f32):
    u = lax.bitcast_convert_type(x_f32, jnp.uint32)
    return (u + 0x7FFF + ((u >> 16) & 1)) >> 16


def _proj_body(w_ref, emb_ref, out_ref):
    res = lax.dot_general(w_ref[...], emb_ref[...],
                          (((0,), (1,)), ((), ())),
                          preferred_element_type=jnp.float32)
    lo = _rne_bf16_bits(res[0:1, :])
    hi = _rne_bf16_bits(res[1:2, :]) << 16
    out_ref[...] = (lo | hi).astype(jnp.int32)


def _project_table(emb_table, W):
    grid = (pl.cdiv(VOCAB, _PROJ_BLK),)
    return pl.pallas_call(
        _proj_body,
        grid=grid,
        in_specs=[
            pl.BlockSpec((HIDDEN, NUM_LABELS), lambda i: (0, 0)),
            pl.BlockSpec((_PROJ_BLK, HIDDEN), lambda i: (i, 0)),
        ],
        out_specs=pl.BlockSpec((1, _PROJ_BLK), lambda i: (0, i)),
        out_shape=jax.ShapeDtypeStruct((1, VOCAB), jnp.int32),
    )(W, emb_table)


def _lane_bcast(vec, lane):
    idx = jnp.full((L, 1), lane, jnp.int32)
    dnums = lax.GatherDimensionNumbers(
        offset_dims=(), collapsed_slice_dims=(0,), start_index_map=(0,))
    return lax.gather(vec, idx, dnums, slice_sizes=(1,),
                      mode=lax.GatherScatterMode.PROMISE_IN_BOUNDS)


def _sc_body(ids_hbm, mask_hbm, tab_hbm, bias_hbm, out_hbm,
             tabv, idsv, maskv, outv, biasv, sem_tab, sem_io):
    wid = lax.axis_index("s") * NC + lax.axis_index("c")
    row0 = wid * ROWS_PER_W

    half = ROWS_PER_W // 2
    tab_dma = pltpu.async_copy(tab_hbm, tabv, sem_tab)
    io_dmas = []
    for h in range(2):
        io_dmas.append(pltpu.async_copy(
            ids_hbm.at[pl.ds(row0 + h * half, half)],
            idsv.at[pl.ds(h * half, half)], sem_io))
        io_dmas.append(pltpu.async_copy(
            mask_hbm.at[pl.ds(row0 + h * half, half)],
            maskv.at[pl.ds(h * half, half)], sem_io))
    pltpu.sync_copy(bias_hbm, biasv)
    bias_vec = biasv[...]
    b0 = _lane_bcast(bias_vec, 0)
    b1 = _lane_bcast(bias_vec, 1)

    zcol = jnp.full((L,), ZCOL, jnp.int32)
    himask = jnp.full((L,), -65536, jnp.int32)
    iota = lax.iota(jnp.int32, L)
    rbase = [iota + lg * L for lg in range(LG)]
    col0 = jnp.zeros((L,), jnp.int32)
    col1 = jnp.ones((L,), jnp.int32)
    zero = jnp.zeros((L,), jnp.float32)

    tab_dma.wait()
    io_dmas[0].wait()
    io_dmas[1].wait()

    for lg in range(LG):
        if lg == LG // 2:
            io_dmas[2].wait()
            io_dmas[3].wait()
        def body(si, accs):
            a0, a1, a2 = accs
            csi = jnp.full((L,), si, jnp.int32)
            ids16 = plsc.load_gather(idsv, [rbase[lg], csi])
            m16 = plsc.load_gather(maskv, [rbase[lg], csi])
            sel = jnp.where(m16 != 0, ids16, zcol)
            g = plsc.load_gather(tabv, [sel])
            g0 = plsc.bitcast(g << 16, jnp.float32)
            g1 = plsc.bitcast(g & himask, jnp.float32)
            return (a0 + g0, a1 + g1, a2 + m16.astype(jnp.float32))
        a0, a1, a2 = lax.fori_loop(0, SEQ, body, (zero, zero, zero),
                                   unroll=8)
        plsc.store_scatter(outv, [rbase[lg], col0], a0 / a2 + b0)
        plsc.store_scatter(outv, [rbase[lg], col1], a1 / a2 + b1)
    pltpu.sync_copy(outv, out_hbm.at[pl.ds(row0, ROWS_PER_W)])


def _sc_pool(ids, mask, tab_flat, bias_pad):
    mesh = plsc.VectorSubcoreMesh(core_axis_name="c", subcore_axis_name="s",
                                  num_cores=NC, num_subcores=NS)
    f = pl.kernel(
        _sc_body,
        out_type=jax.ShapeDtypeStruct((BATCH, NUM_LABELS), jnp.float32),
        mesh=mesh,
        scratch_types=[
            pltpu.VMEM((VP,), jnp.int32),
            pltpu.VMEM((ROWS_PER_W, SEQ), jnp.int32),
            pltpu.VMEM((ROWS_PER_W, SEQ), jnp.int32),
            pltpu.VMEM((ROWS_PER_W, NUM_LABELS), jnp.float32),
            pltpu.VMEM((L,), jnp.float32),
            pltpu.SemaphoreType.DMA,
            pltpu.SemaphoreType.DMA,
        ],
        compiler_params=pltpu.CompilerParams(use_tc_tiling_on_sc=False,
                                             needs_layout_passes=False),
    )
    return f(ids, mask, tab_flat, bias_pad)


@jax.jit
def kernel(input_ids, attention_mask, emb_table, W, b):
    proj = _project_table(emb_table.astype(jnp.float32),
                          W.astype(jnp.float32))
    tab_flat = jnp.pad(proj, ((0, 0), (0, VP - VOCAB))).reshape(-1)
    bias_pad = jnp.pad(b.astype(jnp.float32), (0, L - NUM_LABELS))

    return _sc_pool(input_ids.astype(jnp.int32),
                    attention_mask.astype(jnp.int32), tab_flat, bias_pad)
